# baseline (speedup 1.0000x reference)
import functools

import jax
import jax.numpy as jnp
from jax import lax
from jax.experimental import pallas as pl
from jax.experimental.pallas import tpu as pltpu
from jax.experimental.pallas import tpu_sc as plsc

_B, _N, _D = 4, 8192, 768
_BT = _B * _N
_NW = 32
_BPW = _BT // _NW
_C = 64
_NCHUNK = _BPW // _C

_CH = 1024
_NBLK = _BT // _CH
_R, _CC = 256, 128
_RPB = _N // _CC
_TARGET_RATIOS = (0.1, 0.02)


def _bf16_round(v):
    u = plsc.bitcast(v, jnp.uint32)
    lsb = jnp.right_shift(u, jnp.uint32(16)) & jnp.uint32(1)
    r = u + (jnp.uint32(0x7FFF) + lsb)
    r = r & jnp.uint32(0xFFFF0000)
    return plsc.bitcast(r, jnp.float32)


def _sc_gather_kernel(
    table_hbm, idx_hbm, w1_hbm, w2_hbm,
    out_hbm, l1_hbm, l2_hbm,
    idx_v, buf, w1_v, w2_v, log1_v, log2_v, sem0, sem1, semo,
):
    wid = lax.axis_index("s") * 2 + lax.axis_index("c")
    base = wid * _BPW
    pltpu.sync_copy(idx_hbm.at[pl.ds(base, _BPW)], idx_v)
    pltpu.sync_copy(w1_hbm, w1_v)
    pltpu.sync_copy(w2_hbm, w2_v)
    sems = (sem0, sem1)
    lane = lax.iota(jnp.int32, 16)
    rowvs = [16 * g + lane for g in range(_C // 16)]
    zero16 = jnp.zeros((16,), jnp.float32)

    for t in range(2):
        pltpu.async_copy(
            table_hbm.at[idx_v.at[pl.ds(t * _C, _C)]], buf.at[t], sems[t]
        )

    def body(i, carry):
        for t in range(2):
            c = i * 2 + t
            pltpu.make_async_copy(
                table_hbm.at[pl.ds(0, _C)], buf.at[t], sems[t]
            ).wait()
            pltpu.async_copy(
                buf.at[t], out_hbm.at[pl.ds(base + c * _C, _C)], semo
            )

            bufT = buf.at[t]

            def kbody(k, accs):
                w1blk = _bf16_round(w1_v[pl.ds(16 * k, 16)])
                w2blk = _bf16_round(w2_v[pl.ds(16 * k, 16)])
                for j in range(16):
                    d = 16 * k + j
                    colv = jnp.full((16,), d, jnp.int32)
                    new = []
                    for g in range(_C // 16):
                        xs = _bf16_round(
                            plsc.load_gather(bufT, [rowvs[g], colv])
                        )
                        new.append(accs[2 * g] + xs * w1blk[j])
                        new.append(accs[2 * g + 1] + xs * w2blk[j])
                    accs = tuple(new)
                return accs

            accs = plsc.parallel_loop(
                0, _D // 16, carry=tuple([zero16] * (2 * (_C // 16)))
            )(kbody)
            for g in range(_C // 16):
                log1_v[pl.ds(c * _C + 16 * g, 16)] = accs[2 * g]
                log2_v[pl.ds(c * _C + 16 * g, 16)] = accs[2 * g + 1]

            pltpu.make_async_copy(
                table_hbm.at[pl.ds(0, _C)], buf.at[t], semo
            ).wait()
            nxt = c + 2

            @pl.when(nxt < _NCHUNK)
            def _():
                pltpu.async_copy(
                    table_hbm.at[idx_v.at[pl.ds(nxt * _C, _C)]],
                    buf.at[t],
                    sems[t],
                )
        return carry

    lax.fori_loop(0, _NCHUNK // 2, body, 0)
    pltpu.sync_copy(log1_v, l1_hbm.at[pl.ds(base, _BPW)])
    pltpu.sync_copy(log2_v, l2_hbm.at[pl.ds(base, _BPW)])


@functools.cache
def _sc_gather():
    return pl.kernel(
        _sc_gather_kernel,
        out_type=(
            jax.ShapeDtypeStruct((_BT, _D), jnp.float32),
            jax.ShapeDtypeStruct((_BT,), jnp.float32),
            jax.ShapeDtypeStruct((_BT,), jnp.float32),
        ),
        mesh=plsc.VectorSubcoreMesh(core_axis_name="c", subcore_axis_name="s"),
        compiler_params=pltpu.CompilerParams(
            use_tc_tiling_on_sc=False, needs_layout_passes=False
        ),
        scratch_types=[
            pltpu.VMEM((_BPW,), jnp.int32),
            pltpu.VMEM((2, _C, _D), jnp.float32),
            pltpu.VMEM((_D,), jnp.float32),
            pltpu.VMEM((_D,), jnp.float32),
            pltpu.VMEM((_BPW,), jnp.float32),
            pltpu.VMEM((_BPW,), jnp.float32),
            pltpu.SemaphoreType.DMA,
            pltpu.SemaphoreType.DMA,
            pltpu.SemaphoreType.DMA,
        ],
    )


def _tc_finish_body(
    l1_ref, l2_ref, m_ref, b_ref,
    lid1_ref, hd1_ref, lid2_ref, hd2_ref, loss_ref,
):
    l1 = l1_ref[...] + b_ref[0]
    l2 = l2_ref[...] + b_ref[1]
    maskf = m_ref[...].astype(jnp.float32)
    p1 = jax.nn.sigmoid(l1) * maskf
    p2 = jax.nn.sigmoid(l2) * maskf
    h1 = (p1 > 0.5).astype(jnp.float32)
    h2 = (p2 > 0.5).astype(jnp.float32)

    rowc = lax.broadcasted_iota(jnp.int32, (_CC, _CC), 0)
    colc = lax.broadcasted_iota(jnp.int32, (_CC, _CC), 1)
    upper = (rowc <= colc).astype(jnp.float32)
    win1 = jnp.dot(h1, upper, preferred_element_type=jnp.float32)
    win2 = jnp.dot(h2, upper, preferred_element_type=jnp.float32)

    rowr = lax.broadcasted_iota(jnp.int32, (_R, _R), 0)
    colr = lax.broadcasted_iota(jnp.int32, (_R, _R), 1)
    batch_start = (rowr // _RPB) * _RPB
    wb_lower = jnp.logical_and(colr < rowr, colr >= batch_start)
    wb_lower = wb_lower.astype(jnp.float32)
    rs1 = win1[:, _CC - 1 : _CC]
    rs2 = win2[:, _CC - 1 : _CC]
    cs1 = win1 + jnp.dot(wb_lower, rs1, preferred_element_type=jnp.float32)
    cs2 = win2 + jnp.dot(wb_lower, rs2, preferred_element_type=jnp.float32)

    lid1_ref[...] = cs1.astype(jnp.int32)
    hd1_ref[...] = h1.astype(jnp.int32)
    lid2_ref[...] = cs2.astype(jnp.int32)
    hd2_ref[...] = h2.astype(jnp.int32)

    denom = jnp.maximum(jnp.sum(maskf), 1.0)
    r1 = jnp.sum(p1) / denom
    r2 = jnp.sum(p2) / denom
    loss_ref[0, 0] = (
        (r1 - _TARGET_RATIOS[0]) ** 2 + (r2 - _TARGET_RATIOS[1]) ** 2
    )


def _tc_finish(l1, l2, mv, bc):
    return pl.pallas_call(
        _tc_finish_body,
        in_specs=[
            pl.BlockSpec((_R, _CC), lambda: (0, 0)),
            pl.BlockSpec((_R, _CC), lambda: (0, 0)),
            pl.BlockSpec((_R, _CC), lambda: (0, 0)),
            pl.BlockSpec(memory_space=pltpu.SMEM),
        ],
        out_specs=[
            pl.BlockSpec((_R, _CC), lambda: (0, 0)),
            pl.BlockSpec((_R, _CC), lambda: (0, 0)),
            pl.BlockSpec((_R, _CC), lambda: (0, 0)),
            pl.BlockSpec((_R, _CC), lambda: (0, 0)),
            pl.BlockSpec(memory_space=pltpu.SMEM),
        ],
        out_shape=[
            jax.ShapeDtypeStruct((_R, _CC), jnp.int32),
            jax.ShapeDtypeStruct((_R, _CC), jnp.int32),
            jax.ShapeDtypeStruct((_R, _CC), jnp.int32),
            jax.ShapeDtypeStruct((_R, _CC), jnp.int32),
            jax.ShapeDtypeStruct((1, 1), jnp.float32),
        ],
    )(l1, l2, mv, bc)


def kernel(input_ids, attention_mask, token_embed, W_r1, b_r1, W_r2, b_r2):
    B, N = input_ids.shape
    V, D = token_embed.shape
    assert (B, N, D) == (_B, _N, _D)

    idx = input_ids.reshape(_BT)
    x2, l1, l2 = _sc_gather()(
        token_embed, idx, W_r1.reshape(_D), W_r2.reshape(_D)
    )

    bc = jnp.concatenate([b_r1, b_r2])
    mv = attention_mask.reshape(_R, _CC)
    lid1, hd1, lid2, hd2, loss = _tc_finish(
        l1.reshape(_R, _CC), l2.reshape(_R, _CC), mv, bc
    )

    x = x2.reshape(B, N, D)
    return (
        x,
        lid1.reshape(B, N),
        hd1.reshape(B, N).astype(bool),
        lid2.reshape(B, N),
        hd2.reshape(B, N).astype(bool),
        loss[0, 0],
    )


# speedup vs baseline: 4.5501x; 4.5501x over previous
import functools

import jax
import jax.numpy as jnp
from jax import lax
from jax.experimental import pallas as pl
from jax.experimental.pallas import tpu as pltpu
from jax.experimental.pallas import tpu_sc as plsc

_B, _N, _D = 4, 8192, 768
_BT = _B * _N
_NW = 32
_BPW = _BT // _NW
_C = 64
_NCHUNK = _BPW // _C

_CH = 1024
_NBLK = _BT // _CH
_R, _CC = 256, 128
_RPB = _N // _CC
_TARGET_RATIOS = (0.1, 0.02)


@functools.cache
def _sc_gather(n_tokens):
    bpw = n_tokens // _NW
    nchunk = bpw // _C

    def _sc_gather_kernel(table_hbm, idx_hbm, out_hbm, idx_v, buf, sem0, sem1):
        wid = lax.axis_index("s") * 2 + lax.axis_index("c")
        base = wid * bpw
        pltpu.sync_copy(idx_hbm.at[pl.ds(base, bpw)], idx_v)
        sems = (sem0, sem1)

        for t in range(2):
            pltpu.async_copy(
                table_hbm.at[idx_v.at[pl.ds(t * _C, _C)]], buf.at[t], sems[t]
            )

        def body(i, carry):
            for t in range(2):
                c = i * 2 + t
                pltpu.make_async_copy(
                    table_hbm.at[pl.ds(0, _C)], buf.at[t], sems[t]
                ).wait()
                pltpu.sync_copy(
                    buf.at[t], out_hbm.at[pl.ds(base + c * _C, _C)]
                )
                nxt = c + 2

                @pl.when(nxt < nchunk)
                def _():
                    pltpu.async_copy(
                        table_hbm.at[idx_v.at[pl.ds(nxt * _C, _C)]],
                        buf.at[t],
                        sems[t],
                    )
            return carry

        lax.fori_loop(0, nchunk // 2, body, 0)

    return pl.kernel(
        _sc_gather_kernel,
        out_type=jax.ShapeDtypeStruct((n_tokens, _D), jnp.float32),
        mesh=plsc.VectorSubcoreMesh(core_axis_name="c", subcore_axis_name="s"),
        scratch_types=[
            pltpu.VMEM((bpw,), jnp.int32),
            pltpu.VMEM((2, _C, _D), jnp.float32),
            pltpu.SemaphoreType.DMA,
            pltpu.SemaphoreType.DMA,
        ],
    )


def _tc_logits_body(x_ref, w_ref, l1_ref, l2_ref):
    lg = jnp.dot(x_ref[...], w_ref[...], preferred_element_type=jnp.float32)
    l1_ref[...] = lg[:, 0:1]
    l2_ref[...] = lg[:, 1:2]


def _tc_logits(x2, wc):
    n = x2.shape[0]
    return pl.pallas_call(
        _tc_logits_body,
        grid=(n // _CH,),
        in_specs=[
            pl.BlockSpec((_CH, _D), lambda i: (i, 0)),
            pl.BlockSpec((_D, 2), lambda i: (0, 0)),
        ],
        out_specs=[
            pl.BlockSpec((_CH, 1), lambda i: (i, 0)),
            pl.BlockSpec((_CH, 1), lambda i: (i, 0)),
        ],
        out_shape=[
            jax.ShapeDtypeStruct((n, 1), jnp.float32),
            jax.ShapeDtypeStruct((n, 1), jnp.float32),
        ],
    )(x2, wc)


def _tc_finish_body(
    l1_ref, l2_ref, m_ref, b_ref,
    lid1_ref, hd1_ref, lid2_ref, hd2_ref, loss_ref,
):
    l1 = l1_ref[...] + b_ref[0]
    l2 = l2_ref[...] + b_ref[1]
    maskf = m_ref[...].astype(jnp.float32)
    p1 = jax.nn.sigmoid(l1) * maskf
    p2 = jax.nn.sigmoid(l2) * maskf
    h1 = (p1 > 0.5).astype(jnp.float32)
    h2 = (p2 > 0.5).astype(jnp.float32)

    rowc = lax.broadcasted_iota(jnp.int32, (_CC, _CC), 0)
    colc = lax.broadcasted_iota(jnp.int32, (_CC, _CC), 1)
    upper = (rowc <= colc).astype(jnp.float32)
    win1 = jnp.dot(h1, upper, preferred_element_type=jnp.float32)
    win2 = jnp.dot(h2, upper, preferred_element_type=jnp.float32)

    rowr = lax.broadcasted_iota(jnp.int32, (_R, _R), 0)
    colr = lax.broadcasted_iota(jnp.int32, (_R, _R), 1)
    batch_start = (rowr // _RPB) * _RPB
    wb_lower = jnp.logical_and(colr < rowr, colr >= batch_start)
    wb_lower = wb_lower.astype(jnp.float32)
    rs1 = win1[:, _CC - 1 : _CC]
    rs2 = win2[:, _CC - 1 : _CC]
    cs1 = win1 + jnp.dot(wb_lower, rs1, preferred_element_type=jnp.float32)
    cs2 = win2 + jnp.dot(wb_lower, rs2, preferred_element_type=jnp.float32)

    lid1_ref[...] = cs1.astype(jnp.int32)
    hd1_ref[...] = h1.astype(jnp.int32)
    lid2_ref[...] = cs2.astype(jnp.int32)
    hd2_ref[...] = h2.astype(jnp.int32)

    denom = jnp.maximum(jnp.sum(maskf), 1.0)
    r1 = jnp.sum(p1) / denom
    r2 = jnp.sum(p2) / denom
    loss_ref[0, 0] = (
        (r1 - _TARGET_RATIOS[0]) ** 2 + (r2 - _TARGET_RATIOS[1]) ** 2
    )


def _tc_finish(l1, l2, mv, bc):
    return pl.pallas_call(
        _tc_finish_body,
        in_specs=[
            pl.BlockSpec((_R, _CC), lambda: (0, 0)),
            pl.BlockSpec((_R, _CC), lambda: (0, 0)),
            pl.BlockSpec((_R, _CC), lambda: (0, 0)),
            pl.BlockSpec(memory_space=pltpu.SMEM),
        ],
        out_specs=[
            pl.BlockSpec((_R, _CC), lambda: (0, 0)),
            pl.BlockSpec((_R, _CC), lambda: (0, 0)),
            pl.BlockSpec((_R, _CC), lambda: (0, 0)),
            pl.BlockSpec((_R, _CC), lambda: (0, 0)),
            pl.BlockSpec(memory_space=pltpu.SMEM),
        ],
        out_shape=[
            jax.ShapeDtypeStruct((_R, _CC), jnp.int32),
            jax.ShapeDtypeStruct((_R, _CC), jnp.int32),
            jax.ShapeDtypeStruct((_R, _CC), jnp.int32),
            jax.ShapeDtypeStruct((_R, _CC), jnp.int32),
            jax.ShapeDtypeStruct((1, 1), jnp.float32),
        ],
    )(l1, l2, mv, bc)


def kernel(input_ids, attention_mask, token_embed, W_r1, b_r1, W_r2, b_r2):
    B, N = input_ids.shape
    V, D = token_embed.shape
    assert (B, N, D) == (_B, _N, _D)

    idx = input_ids.reshape(_BT)
    wc = jnp.concatenate([W_r1, W_r2], axis=1)
    bc = jnp.concatenate([b_r1, b_r2])

    nseg = 4
    seg = _BT // nseg
    xs, l1s, l2s = [], [], []
    for s in range(nseg):
        x_s = _sc_gather(seg)(token_embed, lax.slice(idx, (s * seg,), ((s + 1) * seg,)))
        l1_s, l2_s = _tc_logits(x_s, wc)
        xs.append(x_s)
        l1s.append(l1_s)
        l2s.append(l2_s)
    x2 = jnp.concatenate(xs, axis=0)
    l1 = jnp.concatenate(l1s, axis=0)
    l2 = jnp.concatenate(l2s, axis=0)
    mv = attention_mask.reshape(_R, _CC)
    lid1, hd1, lid2, hd2, loss = _tc_finish(
        l1.reshape(_R, _CC), l2.reshape(_R, _CC), mv, bc
    )

    x = x2.reshape(B, N, D)
    return (
        x,
        lid1.reshape(B, N),
        hd1.reshape(B, N).astype(bool),
        lid2.reshape(B, N),
        hd2.reshape(B, N).astype(bool),
        loss[0, 0],
    )


# speedup vs baseline: 7.1427x; 1.5698x over previous
import functools

import jax
import jax.numpy as jnp
from jax import lax
from jax.experimental import pallas as pl
from jax.experimental.pallas import tpu as pltpu
from jax.experimental.pallas import tpu_sc as plsc

_B, _N, _D = 4, 8192, 768
_BT = _B * _N
_NW = 32
_BPW = _BT // _NW
_C = 64
_NCHUNK = _BPW // _C

_CH = 2048
_NBLK = _BT // _CH
_R, _CC = 256, 128
_RPB = _N // _CC
_TARGET_RATIOS = (0.1, 0.02)


def _sc_gather_kernel(table_hbm, idx_hbm, out_hbm, idx_v, buf, sem0, sem1):
    wid = lax.axis_index("s") * 2 + lax.axis_index("c")
    base = wid * _BPW
    pltpu.sync_copy(idx_hbm.at[pl.ds(base, _BPW)], idx_v)
    sems = (sem0, sem1)

    for t in range(2):
        pltpu.async_copy(
            table_hbm.at[idx_v.at[pl.ds(t * _C, _C)]], buf.at[t], sems[t]
        )

    def body(i, carry):
        for t in range(2):
            c = i * 2 + t
            pltpu.make_async_copy(
                table_hbm.at[pl.ds(0, _C)], buf.at[t], sems[t]
            ).wait()
            pltpu.sync_copy(buf.at[t], out_hbm.at[pl.ds(base + c * _C, _C)])
            nxt = c + 2

            @pl.when(nxt < _NCHUNK)
            def _():
                pltpu.async_copy(
                    table_hbm.at[idx_v.at[pl.ds(nxt * _C, _C)]],
                    buf.at[t],
                    sems[t],
                )
        return carry

    lax.fori_loop(0, _NCHUNK // 2, body, 0)


@functools.cache
def _sc_gather():
    return pl.kernel(
        _sc_gather_kernel,
        out_type=jax.ShapeDtypeStruct((_BT, _D), jnp.float32),
        mesh=plsc.VectorSubcoreMesh(core_axis_name="c", subcore_axis_name="s"),
        scratch_types=[
            pltpu.VMEM((_BPW,), jnp.int32),
            pltpu.VMEM((2, _C, _D), jnp.float32),
            pltpu.SemaphoreType.DMA,
            pltpu.SemaphoreType.DMA,
        ],
    )


def _tc_logits_body(x_ref, w_ref, l1_ref, l2_ref):
    lg = jnp.dot(x_ref[...], w_ref[...], preferred_element_type=jnp.float32)
    l1_ref[...] = lg[:, 0:1]
    l2_ref[...] = lg[:, 1:2]


def _tc_logits(x2, wc):
    return pl.pallas_call(
        _tc_logits_body,
        grid=(_NBLK,),
        in_specs=[
            pl.BlockSpec((_CH, _D), lambda i: (i, 0)),
            pl.BlockSpec((_D, 2), lambda i: (0, 0)),
        ],
        out_specs=[
            pl.BlockSpec((_CH, 1), lambda i: (i, 0)),
            pl.BlockSpec((_CH, 1), lambda i: (i, 0)),
        ],
        out_shape=[
            jax.ShapeDtypeStruct((_BT, 1), jnp.float32),
            jax.ShapeDtypeStruct((_BT, 1), jnp.float32),
        ],
    )(x2, wc)


def _tc_finish_body(
    l1_ref, l2_ref, m_ref, b_ref,
    lid1_ref, hd1_ref, lid2_ref, hd2_ref, loss_ref,
):
    l1 = l1_ref[...] + b_ref[0]
    l2 = l2_ref[...] + b_ref[1]
    maskf = m_ref[...].astype(jnp.float32)
    p1 = jax.nn.sigmoid(l1) * maskf
    p2 = jax.nn.sigmoid(l2) * maskf
    h1 = (p1 > 0.5).astype(jnp.float32)
    h2 = (p2 > 0.5).astype(jnp.float32)

    rowc = lax.broadcasted_iota(jnp.int32, (_CC, _CC), 0)
    colc = lax.broadcasted_iota(jnp.int32, (_CC, _CC), 1)
    upper = (rowc <= colc).astype(jnp.float32)
    win1 = jnp.dot(h1, upper, preferred_element_type=jnp.float32)
    win2 = jnp.dot(h2, upper, preferred_element_type=jnp.float32)

    rowr = lax.broadcasted_iota(jnp.int32, (_R, _R), 0)
    colr = lax.broadcasted_iota(jnp.int32, (_R, _R), 1)
    batch_start = (rowr // _RPB) * _RPB
    wb_lower = jnp.logical_and(colr < rowr, colr >= batch_start)
    wb_lower = wb_lower.astype(jnp.float32)
    rs1 = win1[:, _CC - 1 : _CC]
    rs2 = win2[:, _CC - 1 : _CC]
    cs1 = win1 + jnp.dot(wb_lower, rs1, preferred_element_type=jnp.float32)
    cs2 = win2 + jnp.dot(wb_lower, rs2, preferred_element_type=jnp.float32)

    lid1_ref[...] = cs1.astype(jnp.int32)
    hd1_ref[...] = h1.astype(jnp.int32)
    lid2_ref[...] = cs2.astype(jnp.int32)
    hd2_ref[...] = h2.astype(jnp.int32)

    denom = jnp.maximum(jnp.sum(maskf), 1.0)
    r1 = jnp.sum(p1) / denom
    r2 = jnp.sum(p2) / denom
    loss_ref[0, 0] = (
        (r1 - _TARGET_RATIOS[0]) ** 2 + (r2 - _TARGET_RATIOS[1]) ** 2
    )


def _tc_finish(l1, l2, mv, bc):
    return pl.pallas_call(
        _tc_finish_body,
        in_specs=[
            pl.BlockSpec((_R, _CC), lambda: (0, 0)),
            pl.BlockSpec((_R, _CC), lambda: (0, 0)),
            pl.BlockSpec((_R, _CC), lambda: (0, 0)),
            pl.BlockSpec(memory_space=pltpu.SMEM),
        ],
        out_specs=[
            pl.BlockSpec((_R, _CC), lambda: (0, 0)),
            pl.BlockSpec((_R, _CC), lambda: (0, 0)),
            pl.BlockSpec((_R, _CC), lambda: (0, 0)),
            pl.BlockSpec((_R, _CC), lambda: (0, 0)),
            pl.BlockSpec(memory_space=pltpu.SMEM),
        ],
        out_shape=[
            jax.ShapeDtypeStruct((_R, _CC), jnp.int32),
            jax.ShapeDtypeStruct((_R, _CC), jnp.int32),
            jax.ShapeDtypeStruct((_R, _CC), jnp.int32),
            jax.ShapeDtypeStruct((_R, _CC), jnp.int32),
            jax.ShapeDtypeStruct((1, 1), jnp.float32),
        ],
    )(l1, l2, mv, bc)


def kernel(input_ids, attention_mask, token_embed, W_r1, b_r1, W_r2, b_r2):
    B, N = input_ids.shape
    V, D = token_embed.shape
    assert (B, N, D) == (_B, _N, _D)

    idx = input_ids.reshape(_BT)
    x2 = _sc_gather()(token_embed, idx)

    wc = jnp.concatenate([W_r1, W_r2], axis=1)
    bc = jnp.concatenate([b_r1, b_r2])
    l1, l2 = _tc_logits(x2, wc)
    mv = attention_mask.reshape(_R, _CC)
    lid1, hd1, lid2, hd2, loss = _tc_finish(
        l1.reshape(_R, _CC), l2.reshape(_R, _CC), mv, bc
    )

    x = x2.reshape(B, N, D)
    return (
        x,
        lid1.reshape(B, N),
        hd1.reshape(B, N).astype(bool),
        lid2.reshape(B, N),
        hd2.reshape(B, N).astype(bool),
        loss[0, 0],
    )


# speedup vs baseline: 7.1967x; 1.0076x over previous
import functools

import jax
import jax.numpy as jnp
from jax import lax
from jax.experimental import pallas as pl
from jax.experimental.pallas import tpu as pltpu
from jax.experimental.pallas import tpu_sc as plsc

_B, _N, _D = 4, 8192, 768
_BT = _B * _N
_NW = 32
_BPW = _BT // _NW
_C = 64
_NCHUNK = _BPW // _C

_CH = 4096
_NBLK = _BT // _CH
_R, _CC = 256, 128
_RPB = _N // _CC
_TARGET_RATIOS = (0.1, 0.02)


def _sc_gather_kernel(table_hbm, idx_hbm, out_hbm, idx_v, buf, sem0, sem1):
    wid = lax.axis_index("s") * 2 + lax.axis_index("c")
    base = wid * _BPW
    pltpu.sync_copy(idx_hbm.at[pl.ds(base, _BPW)], idx_v)
    sems = (sem0, sem1)

    for t in range(2):
        pltpu.async_copy(
            table_hbm.at[idx_v.at[pl.ds(t * _C, _C)]], buf.at[t], sems[t]
        )

    def body(i, carry):
        for t in range(2):
            c = i * 2 + t
            pltpu.make_async_copy(
                table_hbm.at[pl.ds(0, _C)], buf.at[t], sems[t]
            ).wait()
            pltpu.sync_copy(buf.at[t], out_hbm.at[pl.ds(base + c * _C, _C)])
            nxt = c + 2

            @pl.when(nxt < _NCHUNK)
            def _():
                pltpu.async_copy(
                    table_hbm.at[idx_v.at[pl.ds(nxt * _C, _C)]],
                    buf.at[t],
                    sems[t],
                )
        return carry

    lax.fori_loop(0, _NCHUNK // 2, body, 0)


@functools.cache
def _sc_gather():
    return pl.kernel(
        _sc_gather_kernel,
        out_type=jax.ShapeDtypeStruct((_BT, _D), jnp.float32),
        mesh=plsc.VectorSubcoreMesh(core_axis_name="c", subcore_axis_name="s"),
        scratch_types=[
            pltpu.VMEM((_BPW,), jnp.int32),
            pltpu.VMEM((2, _C, _D), jnp.float32),
            pltpu.SemaphoreType.DMA,
            pltpu.SemaphoreType.DMA,
        ],
    )


def _tc_logits_body(x_ref, w_ref, l1_ref, l2_ref):
    lg = jnp.dot(x_ref[...], w_ref[...], preferred_element_type=jnp.float32)
    l1_ref[...] = lg[:, 0:1]
    l2_ref[...] = lg[:, 1:2]


def _tc_logits(x2, wc):
    return pl.pallas_call(
        _tc_logits_body,
        grid=(_NBLK,),
        in_specs=[
            pl.BlockSpec((_CH, _D), lambda i: (i, 0)),
            pl.BlockSpec((_D, 2), lambda i: (0, 0)),
        ],
        out_specs=[
            pl.BlockSpec((_CH, 1), lambda i: (i, 0)),
            pl.BlockSpec((_CH, 1), lambda i: (i, 0)),
        ],
        out_shape=[
            jax.ShapeDtypeStruct((_BT, 1), jnp.float32),
            jax.ShapeDtypeStruct((_BT, 1), jnp.float32),
        ],
    )(x2, wc)


def _tc_finish_body(
    l1_ref, l2_ref, m_ref, b_ref,
    lid1_ref, hd1_ref, lid2_ref, hd2_ref, loss_ref,
):
    l1 = l1_ref[...] + b_ref[0]
    l2 = l2_ref[...] + b_ref[1]
    maskf = m_ref[...].astype(jnp.float32)
    p1 = jax.nn.sigmoid(l1) * maskf
    p2 = jax.nn.sigmoid(l2) * maskf
    h1 = (p1 > 0.5).astype(jnp.float32)
    h2 = (p2 > 0.5).astype(jnp.float32)

    rowc = lax.broadcasted_iota(jnp.int32, (_CC, _CC), 0)
    colc = lax.broadcasted_iota(jnp.int32, (_CC, _CC), 1)
    upper = (rowc <= colc).astype(jnp.float32)
    win1 = jnp.dot(h1, upper, preferred_element_type=jnp.float32)
    win2 = jnp.dot(h2, upper, preferred_element_type=jnp.float32)

    rowr = lax.broadcasted_iota(jnp.int32, (_R, _R), 0)
    colr = lax.broadcasted_iota(jnp.int32, (_R, _R), 1)
    batch_start = (rowr // _RPB) * _RPB
    wb_lower = jnp.logical_and(colr < rowr, colr >= batch_start)
    wb_lower = wb_lower.astype(jnp.float32)
    rs1 = win1[:, _CC - 1 : _CC]
    rs2 = win2[:, _CC - 1 : _CC]
    cs1 = win1 + jnp.dot(wb_lower, rs1, preferred_element_type=jnp.float32)
    cs2 = win2 + jnp.dot(wb_lower, rs2, preferred_element_type=jnp.float32)

    lid1_ref[...] = cs1.astype(jnp.int32)
    hd1_ref[...] = h1.astype(jnp.int32)
    lid2_ref[...] = cs2.astype(jnp.int32)
    hd2_ref[...] = h2.astype(jnp.int32)

    denom = jnp.maximum(jnp.sum(maskf), 1.0)
    r1 = jnp.sum(p1) / denom
    r2 = jnp.sum(p2) / denom
    loss_ref[0, 0] = (
        (r1 - _TARGET_RATIOS[0]) ** 2 + (r2 - _TARGET_RATIOS[1]) ** 2
    )


def _tc_finish(l1, l2, mv, bc):
    return pl.pallas_call(
        _tc_finish_body,
        in_specs=[
            pl.BlockSpec((_R, _CC), lambda: (0, 0)),
            pl.BlockSpec((_R, _CC), lambda: (0, 0)),
            pl.BlockSpec((_R, _CC), lambda: (0, 0)),
            pl.BlockSpec(memory_space=pltpu.SMEM),
        ],
        out_specs=[
            pl.BlockSpec((_R, _CC), lambda: (0, 0)),
            pl.BlockSpec((_R, _CC), lambda: (0, 0)),
            pl.BlockSpec((_R, _CC), lambda: (0, 0)),
            pl.BlockSpec((_R, _CC), lambda: (0, 0)),
            pl.BlockSpec(memory_space=pltpu.SMEM),
        ],
        out_shape=[
            jax.ShapeDtypeStruct((_R, _CC), jnp.int32),
            jax.ShapeDtypeStruct((_R, _CC), jnp.int32),
            jax.ShapeDtypeStruct((_R, _CC), jnp.int32),
            jax.ShapeDtypeStruct((_R, _CC), jnp.int32),
            jax.ShapeDtypeStruct((1, 1), jnp.float32),
        ],
    )(l1, l2, mv, bc)


def kernel(input_ids, attention_mask, token_embed, W_r1, b_r1, W_r2, b_r2):
    B, N = input_ids.shape
    V, D = token_embed.shape
    assert (B, N, D) == (_B, _N, _D)

    idx = input_ids.reshape(_BT)
    x2 = _sc_gather()(token_embed, idx)

    wc = jnp.concatenate([W_r1, W_r2], axis=1)
    bc = jnp.concatenate([b_r1, b_r2])
    l1, l2 = _tc_logits(x2, wc)
    mv = attention_mask.reshape(_R, _CC)
    lid1, hd1, lid2, hd2, loss = _tc_finish(
        l1.reshape(_R, _CC), l2.reshape(_R, _CC), mv, bc
    )

    x = x2.reshape(B, N, D)
    return (
        x,
        lid1.reshape(B, N),
        hd1.reshape(B, N).astype(bool),
        lid2.reshape(B, N),
        hd2.reshape(B, N).astype(bool),
        loss[0, 0],
    )


# speedup vs baseline: 7.2652x; 1.0095x over previous
import functools

import jax
import jax.numpy as jnp
from jax import lax
from jax.experimental import pallas as pl
from jax.experimental.pallas import tpu as pltpu
from jax.experimental.pallas import tpu_sc as plsc

_B, _N, _D = 4, 8192, 768
_BT = _B * _N
_NW = 32
_BPW = _BT // _NW
_C = 32
_NCHUNK = _BPW // _C
_NS = 4

_CH = 4096
_NBLK = _BT // _CH
_R, _CC = 256, 128
_RPB = _N // _CC
_TARGET_RATIOS = (0.1, 0.02)


def _sc_gather_kernel(
    table_hbm, idx_hbm, out_hbm, idx_v, buf,
    g0, g1, g2, g3, o0, o1, o2, o3,
):
    wid = lax.axis_index("s") * 2 + lax.axis_index("c")
    base = wid * _BPW
    pltpu.sync_copy(idx_hbm.at[pl.ds(base, _BPW)], idx_v)
    semg = (g0, g1, g2, g3)
    semo = (o0, o1, o2, o3)

    def _issue_gather(c, t):
        pltpu.async_copy(
            table_hbm.at[idx_v.at[pl.ds(c * _C, _C)]], buf.at[t], semg[t]
        )

    def _drain(sem, t):
        pltpu.make_async_copy(table_hbm.at[pl.ds(0, _C)], buf.at[t], sem).wait()

    for t in range(2):
        _issue_gather(t, t)

    def body(i, carry):
        for t in range(_NS):
            c = i * _NS + t
            _drain(semg[t], t)
            pltpu.async_copy(
                buf.at[t], out_hbm.at[pl.ds(base + c * _C, _C)], semo[t]
            )
            t2 = (t + 2) % _NS

            @pl.when(c >= 2)
            def _():
                _drain(semo[t2], t2)

            @pl.when(c + 2 < _NCHUNK)
            def _():
                _issue_gather(c + 2, t2)
        return carry

    lax.fori_loop(0, _NCHUNK // _NS, body, 0)
    _drain(semo[(_NCHUNK - 2) % _NS], (_NCHUNK - 2) % _NS)
    _drain(semo[(_NCHUNK - 1) % _NS], (_NCHUNK - 1) % _NS)


@functools.cache
def _sc_gather():
    return pl.kernel(
        _sc_gather_kernel,
        out_type=jax.ShapeDtypeStruct((_BT, _D), jnp.float32),
        mesh=plsc.VectorSubcoreMesh(core_axis_name="c", subcore_axis_name="s"),
        scratch_types=[
            pltpu.VMEM((_BPW,), jnp.int32),
            pltpu.VMEM((_NS, _C, _D), jnp.float32),
            pltpu.SemaphoreType.DMA,
            pltpu.SemaphoreType.DMA,
            pltpu.SemaphoreType.DMA,
            pltpu.SemaphoreType.DMA,
            pltpu.SemaphoreType.DMA,
            pltpu.SemaphoreType.DMA,
            pltpu.SemaphoreType.DMA,
            pltpu.SemaphoreType.DMA,
        ],
    )


def _tc_logits_body(x_ref, w_ref, l1_ref, l2_ref):
    lg = jnp.dot(x_ref[...], w_ref[...], preferred_element_type=jnp.float32)
    l1_ref[...] = lg[:, 0:1]
    l2_ref[...] = lg[:, 1:2]


def _tc_logits(x2, wc):
    return pl.pallas_call(
        _tc_logits_body,
        grid=(_NBLK,),
        in_specs=[
            pl.BlockSpec((_CH, _D), lambda i: (i, 0)),
            pl.BlockSpec((_D, 2), lambda i: (0, 0)),
        ],
        out_specs=[
            pl.BlockSpec((_CH, 1), lambda i: (i, 0)),
            pl.BlockSpec((_CH, 1), lambda i: (i, 0)),
        ],
        out_shape=[
            jax.ShapeDtypeStruct((_BT, 1), jnp.float32),
            jax.ShapeDtypeStruct((_BT, 1), jnp.float32),
        ],
    )(x2, wc)


def _tc_finish_body(
    l1_ref, l2_ref, m_ref, b_ref,
    lid1_ref, hd1_ref, lid2_ref, hd2_ref, loss_ref,
):
    l1 = l1_ref[...] + b_ref[0]
    l2 = l2_ref[...] + b_ref[1]
    maskf = m_ref[...].astype(jnp.float32)
    p1 = jax.nn.sigmoid(l1) * maskf
    p2 = jax.nn.sigmoid(l2) * maskf
    h1 = (p1 > 0.5).astype(jnp.float32)
    h2 = (p2 > 0.5).astype(jnp.float32)

    rowc = lax.broadcasted_iota(jnp.int32, (_CC, _CC), 0)
    colc = lax.broadcasted_iota(jnp.int32, (_CC, _CC), 1)
    upper = (rowc <= colc).astype(jnp.float32)
    win1 = jnp.dot(h1, upper, preferred_element_type=jnp.float32)
    win2 = jnp.dot(h2, upper, preferred_element_type=jnp.float32)

    rowr = lax.broadcasted_iota(jnp.int32, (_R, _R), 0)
    colr = lax.broadcasted_iota(jnp.int32, (_R, _R), 1)
    batch_start = (rowr // _RPB) * _RPB
    wb_lower = jnp.logical_and(colr < rowr, colr >= batch_start)
    wb_lower = wb_lower.astype(jnp.float32)
    rs1 = win1[:, _CC - 1 : _CC]
    rs2 = win2[:, _CC - 1 : _CC]
    cs1 = win1 + jnp.dot(wb_lower, rs1, preferred_element_type=jnp.float32)
    cs2 = win2 + jnp.dot(wb_lower, rs2, preferred_element_type=jnp.float32)

    lid1_ref[...] = cs1.astype(jnp.int32)
    hd1_ref[...] = h1.astype(jnp.int32)
    lid2_ref[...] = cs2.astype(jnp.int32)
    hd2_ref[...] = h2.astype(jnp.int32)

    denom = jnp.maximum(jnp.sum(maskf), 1.0)
    r1 = jnp.sum(p1) / denom
    r2 = jnp.sum(p2) / denom
    loss_ref[0, 0] = (
        (r1 - _TARGET_RATIOS[0]) ** 2 + (r2 - _TARGET_RATIOS[1]) ** 2
    )


def _tc_finish(l1, l2, mv, bc):
    return pl.pallas_call(
        _tc_finish_body,
        in_specs=[
            pl.BlockSpec((_R, _CC), lambda: (0, 0)),
            pl.BlockSpec((_R, _CC), lambda: (0, 0)),
            pl.BlockSpec((_R, _CC), lambda: (0, 0)),
            pl.BlockSpec(memory_space=pltpu.SMEM),
        ],
        out_specs=[
            pl.BlockSpec((_R, _CC), lambda: (0, 0)),
            pl.BlockSpec((_R, _CC), lambda: (0, 0)),
            pl.BlockSpec((_R, _CC), lambda: (0, 0)),
            pl.BlockSpec((_R, _CC), lambda: (0, 0)),
            pl.BlockSpec(memory_space=pltpu.SMEM),
        ],
        out_shape=[
            jax.ShapeDtypeStruct((_R, _CC), jnp.int32),
            jax.ShapeDtypeStruct((_R, _CC), jnp.int32),
            jax.ShapeDtypeStruct((_R, _CC), jnp.int32),
            jax.ShapeDtypeStruct((_R, _CC), jnp.int32),
            jax.ShapeDtypeStruct((1, 1), jnp.float32),
        ],
    )(l1, l2, mv, bc)


def kernel(input_ids, attention_mask, token_embed, W_r1, b_r1, W_r2, b_r2):
    B, N = input_ids.shape
    V, D = token_embed.shape
    assert (B, N, D) == (_B, _N, _D)

    idx = input_ids.reshape(_BT)
    x2 = _sc_gather()(token_embed, idx)

    wc = jnp.concatenate([W_r1, W_r2], axis=1)
    bc = jnp.concatenate([b_r1, b_r2])
    l1, l2 = _tc_logits(x2, wc)
    mv = attention_mask.reshape(_R, _CC)
    lid1, hd1, lid2, hd2, loss = _tc_finish(
        l1.reshape(_R, _CC), l2.reshape(_R, _CC), mv, bc
    )

    x = x2.reshape(B, N, D)
    return (
        x,
        lid1.reshape(B, N),
        hd1.reshape(B, N).astype(bool),
        lid2.reshape(B, N),
        hd2.reshape(B, N).astype(bool),
        loss[0, 0],
    )
